# baseline (speedup 1.0000x reference)
import functools

import numpy as np
import jax
import jax.numpy as jnp
from jax import lax
from jax.experimental import pallas as pl
from jax.experimental.pallas import tpu as pltpu

_NCB = 4
_K = 8192
_D = 32
_BETA = 0.25
_TEMP = max(1.0 * 0.999, 0.1)
_DIVERSITY_WEIGHT = 1.0

_ROT_A = (13, 15, 26, 6)
_ROT_B = (17, 29, 16, 24)


def _threefry2x32_np(k0, k1, x0, x1):
    k0 = np.uint32(k0)
    k1 = np.uint32(k1)
    x0 = np.asarray(x0, np.uint32).copy()
    x1 = np.asarray(x1, np.uint32).copy()
    ks = [k0, k1, np.uint32(k0 ^ k1 ^ np.uint32(0x1BD11BDA))]

    def rotl(x, d):
        return ((x << np.uint32(d)) | (x >> np.uint32(32 - d))).astype(np.uint32)

    with np.errstate(over="ignore"):
        x0 = (x0 + ks[0]).astype(np.uint32)
        x1 = (x1 + ks[1]).astype(np.uint32)
        for j in range(5):
            for r in (_ROT_A if j % 2 == 0 else _ROT_B):
                x0 = (x0 + x1).astype(np.uint32)
                x1 = rotl(x1, r)
                x1 = (x1 ^ x0).astype(np.uint32)
            x0 = (x0 + ks[(j + 1) % 3]).astype(np.uint32)
            x1 = (x1 + ks[(j + 2) % 3] + np.uint32(j + 1)).astype(np.uint32)
    return x0, x1


def _folded_keys(ncb):
    keys = []
    for i in range(ncb):
        a, b = _threefry2x32_np(0, 1, np.array([0], np.uint32), np.array([i], np.uint32))
        keys.append((int(a[0]), int(b[0])))
    return tuple(keys)


_KEYS = _folded_keys(_NCB)
_TINY = float(np.finfo(np.float32).tiny)


def _tf_bits(p, key):
    k0, k1 = key
    ks = (jnp.uint32(k0), jnp.uint32(k1),
          jnp.uint32((k0 ^ k1 ^ 0x1BD11BDA) & 0xFFFFFFFF))
    x0 = jnp.zeros_like(p) + ks[0]
    x1 = p + ks[1]
    for j in range(5):
        for r in (_ROT_A if j % 2 == 0 else _ROT_B):
            x0 = x0 + x1
            x1 = (x1 << jnp.uint32(r)) | (x1 >> jnp.uint32(32 - r))
            x1 = x1 ^ x0
        x0 = x0 + ks[(j + 1) % 3]
        x1 = x1 + ks[(j + 2) % 3] + jnp.uint32(j + 1)
    return x0 ^ x1


def _gumbel(bits):
    fb = (bits >> jnp.uint32(9)) | jnp.uint32(0x3F800000)
    f = lax.bitcast_convert_type(fb, jnp.float32) - 1.0
    u = jnp.maximum(f, jnp.float32(_TINY))
    return -jnp.log(-jnp.log(u))


_RECIP_TEMP = float(np.float32(1.0) / np.float32(_TEMP))


def _body(x_ref, rot_ref, cb_ref, qsum_ref, idx_ref, *, T, KB, K, NCB):
    i = pl.program_id(0)
    xbf = x_ref[...].astype(jnp.bfloat16)
    rbf16 = rot_ref[...].astype(jnp.bfloat16)
    r = jnp.dot(xbf, rbf16, preferred_element_type=jnp.float32)
    qsum = jnp.zeros_like(r)
    t0 = i * T
    trow = jax.lax.broadcasted_iota(jnp.int32, (T, KB), 0) + t0
    kcol = jax.lax.broadcasted_iota(jnp.int32, (T, KB), 1)
    pbase = (trow * K + kcol).astype(jnp.uint32)
    koh = jax.lax.broadcasted_iota(jnp.int32, (T, K), 1)
    idx_cols = []
    for c in range(NCB):
        cb = cb_ref[c]
        cbbf = cb.astype(jnp.bfloat16)
        sc = jnp.sum(cb * cb, axis=1)
        sr = jnp.sum(r * r, axis=1, keepdims=True)
        rbf = r.astype(jnp.bfloat16)
        rmax = jnp.full((T, 1), -jnp.inf, jnp.float32)
        ridx = jnp.zeros((T, 1), jnp.int32)
        for kc in range(K // KB):
            kb0 = kc * KB
            mm = lax.dot_general(rbf, cbbf[kb0:kb0 + KB],
                                 (((1,), (1,)), ((), ())),
                                 preferred_element_type=jnp.float32)
            d = (sr + sc[kb0:kb0 + KB][None, :]) - 2.0 * mm
            logits = (-d) * jnp.float32(_RECIP_TEMP)
            g = _gumbel(_tf_bits(pbase + jnp.uint32(kb0), _KEYS[c]))
            v = g + logits
            cmax = jnp.max(v, axis=1, keepdims=True)
            cidx = jnp.min(jnp.where(v == cmax, kcol + kb0, K),
                           axis=1, keepdims=True)
            upd = cmax > rmax
            rmax = jnp.where(upd, cmax.astype(jnp.bfloat16).astype(jnp.float32),
                             rmax)
            ridx = jnp.where(upd, cidx, ridx)
        oh = (koh == ridx).astype(jnp.float32)
        q = lax.dot_general(oh, cb, (((1,), (0,)), ((), ())),
                            precision=lax.Precision.HIGHEST,
                            preferred_element_type=jnp.float32)
        qsum = qsum + q
        r = r - q
        idx_cols.append(ridx)
    qsum_ref[...] = qsum
    idx_ref[...] = jnp.concatenate(idx_cols, axis=1)


def _quantize(x2d, rotation, codebooks, interpret=False, T=None, KB=None):
    N, D = x2d.shape
    NCB, K, _ = codebooks.shape
    T = T or min(256, N)
    KB = KB or min(1024, K)
    body = functools.partial(_body, T=T, KB=KB, K=K, NCB=NCB)
    return pl.pallas_call(
        body,
        grid=(N // T,),
        in_specs=[
            pl.BlockSpec((T, D), lambda i: (i, 0)),
            pl.BlockSpec((D, D), lambda i: (0, 0)),
            pl.BlockSpec((NCB, K, D), lambda i: (0, 0, 0)),
        ],
        out_specs=[
            pl.BlockSpec((T, D), lambda i: (i, 0)),
            pl.BlockSpec((T, NCB), lambda i: (i, 0)),
        ],
        out_shape=[
            jax.ShapeDtypeStruct((N, D), jnp.float32),
            jax.ShapeDtypeStruct((N, NCB), jnp.int32),
        ],
        compiler_params=pltpu.CompilerParams(
            dimension_semantics=("arbitrary",)),
        interpret=interpret,
    )(x2d, rotation, codebooks)


def kernel(x, rotation, codebooks):
    B, C, H, W = x.shape
    N = B * H * W
    x2d = jnp.transpose(x, (0, 2, 3, 1)).reshape(N, C)
    qsum, idx2d = _quantize(x2d, rotation, codebooks)
    quant_out = qsum.reshape(x.shape)
    encoding_indices = idx2d.T
    commitment_loss = _BETA * jnp.mean((lax.stop_gradient(quant_out) - x) ** 2)
    codebook_loss = jnp.mean((quant_out - lax.stop_gradient(x)) ** 2)
    idx_cat = encoding_indices.reshape(-1).astype(jnp.float32)
    avg_probs = jnp.mean(idx_cat, axis=0)
    entropy_loss = -jnp.sum(avg_probs * jnp.log(avg_probs + 1e-10)) * _DIVERSITY_WEIGHT
    loss = codebook_loss + commitment_loss + entropy_loss
    quant_out_st = x + lax.stop_gradient(quant_out - x)
    return quant_out_st, loss, encoding_indices


# speedup vs baseline: 1.0001x; 1.0001x over previous
import functools

import numpy as np
import jax
import jax.numpy as jnp
from jax import lax
from jax.experimental import pallas as pl
from jax.experimental.pallas import tpu as pltpu

_NCB = 4
_K = 8192
_D = 32
_BETA = 0.25
_TEMP = max(1.0 * 0.999, 0.1)
_DIVERSITY_WEIGHT = 1.0

_ROT_A = (13, 15, 26, 6)
_ROT_B = (17, 29, 16, 24)


def _threefry2x32_np(k0, k1, x0, x1):
    k0 = np.uint32(k0)
    k1 = np.uint32(k1)
    x0 = np.asarray(x0, np.uint32).copy()
    x1 = np.asarray(x1, np.uint32).copy()
    ks = [k0, k1, np.uint32(k0 ^ k1 ^ np.uint32(0x1BD11BDA))]

    def rotl(x, d):
        return ((x << np.uint32(d)) | (x >> np.uint32(32 - d))).astype(np.uint32)

    with np.errstate(over="ignore"):
        x0 = (x0 + ks[0]).astype(np.uint32)
        x1 = (x1 + ks[1]).astype(np.uint32)
        for j in range(5):
            for r in (_ROT_A if j % 2 == 0 else _ROT_B):
                x0 = (x0 + x1).astype(np.uint32)
                x1 = rotl(x1, r)
                x1 = (x1 ^ x0).astype(np.uint32)
            x0 = (x0 + ks[(j + 1) % 3]).astype(np.uint32)
            x1 = (x1 + ks[(j + 2) % 3] + np.uint32(j + 1)).astype(np.uint32)
    return x0, x1


def _folded_keys(ncb):
    keys = []
    for i in range(ncb):
        a, b = _threefry2x32_np(0, 1, np.array([0], np.uint32), np.array([i], np.uint32))
        keys.append((int(a[0]), int(b[0])))
    return tuple(keys)


_KEYS = _folded_keys(_NCB)
_TINY = float(np.finfo(np.float32).tiny)


def _tf_bits(p, key):
    k0, k1 = key
    ks = (jnp.uint32(k0), jnp.uint32(k1),
          jnp.uint32((k0 ^ k1 ^ 0x1BD11BDA) & 0xFFFFFFFF))
    x0 = jnp.zeros_like(p) + ks[0]
    x1 = p + ks[1]
    for j in range(5):
        for r in (_ROT_A if j % 2 == 0 else _ROT_B):
            x0 = x0 + x1
            x1 = (x1 << jnp.uint32(r)) | (x1 >> jnp.uint32(32 - r))
            x1 = x1 ^ x0
        x0 = x0 + ks[(j + 1) % 3]
        x1 = x1 + ks[(j + 2) % 3] + jnp.uint32(j + 1)
    return x0 ^ x1


def _gumbel(bits):
    fb = (bits >> jnp.uint32(9)) | jnp.uint32(0x3F800000)
    f = lax.bitcast_convert_type(fb, jnp.float32) - 1.0
    u = jnp.maximum(f, jnp.float32(_TINY))
    return -jnp.log(-jnp.log(u))


_RECIP_TEMP = float(np.float32(1.0) / np.float32(_TEMP))


def _body(x_ref, rot_ref, cb_ref, qsum_ref, idx_ref, *, T, KB, K, NCB):
    i = pl.program_id(0)
    xbf = x_ref[...].astype(jnp.bfloat16)
    rbf16 = rot_ref[...].astype(jnp.bfloat16)
    r = jnp.dot(xbf, rbf16, preferred_element_type=jnp.float32)
    qsum = jnp.zeros_like(r)
    t0 = i * T
    trow = jax.lax.broadcasted_iota(jnp.int32, (T, KB), 0) + t0
    kcol = jax.lax.broadcasted_iota(jnp.int32, (T, KB), 1)
    pbase = (trow * K + kcol).astype(jnp.uint32)
    koh = jax.lax.broadcasted_iota(jnp.int32, (T, K), 1)
    idx_cols = []
    for c in range(NCB):
        cb = cb_ref[c]
        cbbf = cb.astype(jnp.bfloat16)
        sc = jnp.sum(cb * cb, axis=1)
        sr = jnp.sum(r * r, axis=1, keepdims=True)
        rbf = r.astype(jnp.bfloat16)
        rmax = jnp.full((T, 1), -jnp.inf, jnp.float32)
        ridx = jnp.zeros((T, 1), jnp.int32)
        for kc in range(K // KB):
            kb0 = kc * KB
            mm = lax.dot_general(rbf, cbbf[kb0:kb0 + KB],
                                 (((1,), (1,)), ((), ())),
                                 preferred_element_type=jnp.float32)
            d = (sr + sc[kb0:kb0 + KB][None, :]) - 2.0 * mm
            logits = (-d) * jnp.float32(_RECIP_TEMP)
            g = _gumbel(_tf_bits(pbase + jnp.uint32(kb0), _KEYS[c]))
            v = g + logits
            cmax = jnp.max(v, axis=1, keepdims=True)
            cidx = jnp.min(jnp.where(v == cmax, kcol + kb0, K),
                           axis=1, keepdims=True)
            upd = cmax > rmax
            rmax = jnp.where(upd, cmax.astype(jnp.bfloat16).astype(jnp.float32),
                             rmax)
            ridx = jnp.where(upd, cidx, ridx)
        oh = (koh == ridx).astype(jnp.float32)
        q = lax.dot_general(oh, cb, (((1,), (0,)), ((), ())),
                            precision=lax.Precision.HIGHEST,
                            preferred_element_type=jnp.float32)
        qsum = qsum + q
        r = r - q
        idx_cols.append(ridx)
    qsum_ref[...] = qsum
    idx_ref[...] = jnp.concatenate(idx_cols, axis=1)


def _quantize(x2d, rotation, codebooks, interpret=False, T=None, KB=None):
    N, D = x2d.shape
    NCB, K, _ = codebooks.shape
    T = T or min(256, N)
    KB = KB or min(1024, K)
    body = functools.partial(_body, T=T, KB=KB, K=K, NCB=NCB)
    return pl.pallas_call(
        body,
        grid=(N // T,),
        in_specs=[
            pl.BlockSpec((T, D), lambda i: (i, 0)),
            pl.BlockSpec((D, D), lambda i: (0, 0)),
            pl.BlockSpec((NCB, K, D), lambda i: (0, 0, 0)),
        ],
        out_specs=[
            pl.BlockSpec((T, D), lambda i: (i, 0)),
            pl.BlockSpec((T, NCB), lambda i: (i, 0)),
        ],
        out_shape=[
            jax.ShapeDtypeStruct((N, D), jnp.float32),
            jax.ShapeDtypeStruct((N, NCB), jnp.int32),
        ],
        compiler_params=pltpu.CompilerParams(
            dimension_semantics=("parallel",)),
        interpret=interpret,
    )(x2d, rotation, codebooks)


def kernel(x, rotation, codebooks):
    B, C, H, W = x.shape
    N = B * H * W
    x2d = jnp.transpose(x, (0, 2, 3, 1)).reshape(N, C)
    qsum, idx2d = _quantize(x2d, rotation, codebooks)
    quant_out = qsum.reshape(x.shape)
    encoding_indices = idx2d.T
    commitment_loss = _BETA * jnp.mean((lax.stop_gradient(quant_out) - x) ** 2)
    codebook_loss = jnp.mean((quant_out - lax.stop_gradient(x)) ** 2)
    idx_cat = encoding_indices.reshape(-1).astype(jnp.float32)
    avg_probs = jnp.mean(idx_cat, axis=0)
    entropy_loss = -jnp.sum(avg_probs * jnp.log(avg_probs + 1e-10)) * _DIVERSITY_WEIGHT
    loss = codebook_loss + commitment_loss + entropy_loss
    quant_out_st = x + lax.stop_gradient(quant_out - x)
    return quant_out_st, loss, encoding_indices


# speedup vs baseline: 1.0372x; 1.0372x over previous
import functools

import numpy as np
import jax
import jax.numpy as jnp
from jax import lax
from jax.experimental import pallas as pl
from jax.experimental.pallas import tpu as pltpu

_NCB = 4
_K = 8192
_D = 32
_BETA = 0.25
_TEMP = max(1.0 * 0.999, 0.1)
_DIVERSITY_WEIGHT = 1.0

_ROT_A = (13, 15, 26, 6)
_ROT_B = (17, 29, 16, 24)


def _threefry2x32_np(k0, k1, x0, x1):
    k0 = np.uint32(k0)
    k1 = np.uint32(k1)
    x0 = np.asarray(x0, np.uint32).copy()
    x1 = np.asarray(x1, np.uint32).copy()
    ks = [k0, k1, np.uint32(k0 ^ k1 ^ np.uint32(0x1BD11BDA))]

    def rotl(x, d):
        return ((x << np.uint32(d)) | (x >> np.uint32(32 - d))).astype(np.uint32)

    with np.errstate(over="ignore"):
        x0 = (x0 + ks[0]).astype(np.uint32)
        x1 = (x1 + ks[1]).astype(np.uint32)
        for j in range(5):
            for r in (_ROT_A if j % 2 == 0 else _ROT_B):
                x0 = (x0 + x1).astype(np.uint32)
                x1 = rotl(x1, r)
                x1 = (x1 ^ x0).astype(np.uint32)
            x0 = (x0 + ks[(j + 1) % 3]).astype(np.uint32)
            x1 = (x1 + ks[(j + 2) % 3] + np.uint32(j + 1)).astype(np.uint32)
    return x0, x1


def _folded_keys(ncb):
    keys = []
    for i in range(ncb):
        a, b = _threefry2x32_np(0, 1, np.array([0], np.uint32), np.array([i], np.uint32))
        keys.append((int(a[0]), int(b[0])))
    return tuple(keys)


_KEYS = _folded_keys(_NCB)
_TINY = float(np.finfo(np.float32).tiny)


def _tf_bits(p, key):
    k0, k1 = key
    ks = (jnp.uint32(k0), jnp.uint32(k1),
          jnp.uint32((k0 ^ k1 ^ 0x1BD11BDA) & 0xFFFFFFFF))
    x0 = jnp.zeros_like(p) + ks[0]
    x1 = p + ks[1]
    for j in range(5):
        for r in (_ROT_A if j % 2 == 0 else _ROT_B):
            x0 = x0 + x1
            x1 = (x1 << jnp.uint32(r)) | (x1 >> jnp.uint32(32 - r))
            x1 = x1 ^ x0
        x0 = x0 + ks[(j + 1) % 3]
        x1 = x1 + ks[(j + 2) % 3] + jnp.uint32(j + 1)
    return x0 ^ x1


def _gumbel(bits):
    fb = (bits >> jnp.uint32(9)) | jnp.uint32(0x3F800000)
    f = lax.bitcast_convert_type(fb, jnp.float32) - 1.0
    u = jnp.maximum(f, jnp.float32(_TINY))
    return -jnp.log(-jnp.log(u))


_RECIP_TEMP = float(np.float32(1.0) / np.float32(_TEMP))


def _body(x_ref, rot_ref, cb_ref, qsum_ref, idx_ref, *, T, KB, K, NCB):
    i = pl.program_id(0)
    xbf = x_ref[...].astype(jnp.bfloat16)
    rbf16 = rot_ref[...].astype(jnp.bfloat16)
    r = jnp.dot(xbf, rbf16, preferred_element_type=jnp.float32)
    qsum = jnp.zeros_like(r)
    t0 = i * T
    trow = jax.lax.broadcasted_iota(jnp.int32, (T, KB), 0) + t0
    kcol = jax.lax.broadcasted_iota(jnp.int32, (T, KB), 1)
    pbase = (trow * K + kcol).astype(jnp.uint32)
    koh = jax.lax.broadcasted_iota(jnp.int32, (T, K), 1)
    idx_cols = []
    for c in range(NCB):
        cb = cb_ref[c]
        cbbf = cb.astype(jnp.bfloat16)
        sc = jnp.sum(cb * cb, axis=1)
        sr = jnp.sum(r * r, axis=1, keepdims=True)
        rbf = r.astype(jnp.bfloat16)
        rmax = jnp.full((T, 1), -jnp.inf, jnp.float32)
        ridx = jnp.zeros((T, 1), jnp.int32)
        for kc in range(K // KB):
            kb0 = kc * KB
            mm = lax.dot_general(rbf, cbbf[kb0:kb0 + KB],
                                 (((1,), (1,)), ((), ())),
                                 preferred_element_type=jnp.float32)
            d = (sr + sc[kb0:kb0 + KB][None, :]) - 2.0 * mm
            logits = (-d) * jnp.float32(_RECIP_TEMP)
            g = _gumbel(_tf_bits(pbase + jnp.uint32(kb0), _KEYS[c]))
            v = g + logits
            cmax = jnp.max(v, axis=1, keepdims=True)
            cidx = jnp.min(jnp.where(v == cmax, kcol + kb0, K),
                           axis=1, keepdims=True)
            upd = cmax > rmax
            rmax = jnp.where(upd, cmax.astype(jnp.bfloat16).astype(jnp.float32),
                             rmax)
            ridx = jnp.where(upd, cidx, ridx)
        oh = (koh == ridx).astype(jnp.float32)
        q = lax.dot_general(oh, cb, (((1,), (0,)), ((), ())),
                            precision=lax.Precision.HIGHEST,
                            preferred_element_type=jnp.float32)
        qsum = qsum + q
        r = r - q
        idx_cols.append(ridx)
    qsum_ref[...] = qsum
    idx_ref[...] = jnp.concatenate(idx_cols, axis=1)


def _quantize(x2d, rotation, codebooks, interpret=False, T=None, KB=None):
    N, D = x2d.shape
    NCB, K, _ = codebooks.shape
    T = T or min(512, N)
    KB = KB or min(1024, K)
    body = functools.partial(_body, T=T, KB=KB, K=K, NCB=NCB)
    return pl.pallas_call(
        body,
        grid=(N // T,),
        in_specs=[
            pl.BlockSpec((T, D), lambda i: (i, 0)),
            pl.BlockSpec((D, D), lambda i: (0, 0)),
            pl.BlockSpec((NCB, K, D), lambda i: (0, 0, 0)),
        ],
        out_specs=[
            pl.BlockSpec((T, D), lambda i: (i, 0)),
            pl.BlockSpec((T, NCB), lambda i: (i, 0)),
        ],
        out_shape=[
            jax.ShapeDtypeStruct((N, D), jnp.float32),
            jax.ShapeDtypeStruct((N, NCB), jnp.int32),
        ],
        compiler_params=pltpu.CompilerParams(
            dimension_semantics=("parallel",)),
        interpret=interpret,
    )(x2d, rotation, codebooks)


def kernel(x, rotation, codebooks):
    B, C, H, W = x.shape
    N = B * H * W
    x2d = jnp.transpose(x, (0, 2, 3, 1)).reshape(N, C)
    qsum, idx2d = _quantize(x2d, rotation, codebooks)
    quant_out = qsum.reshape(x.shape)
    encoding_indices = idx2d.T
    commitment_loss = _BETA * jnp.mean((lax.stop_gradient(quant_out) - x) ** 2)
    codebook_loss = jnp.mean((quant_out - lax.stop_gradient(x)) ** 2)
    idx_cat = encoding_indices.reshape(-1).astype(jnp.float32)
    avg_probs = jnp.mean(idx_cat, axis=0)
    entropy_loss = -jnp.sum(avg_probs * jnp.log(avg_probs + 1e-10)) * _DIVERSITY_WEIGHT
    loss = codebook_loss + commitment_loss + entropy_loss
    quant_out_st = x + lax.stop_gradient(quant_out - x)
    return quant_out_st, loss, encoding_indices
